# baseline (speedup 1.0000x reference)
import jax
import jax.numpy as jnp
from jax.experimental import pallas as pl

B, L, D = 4096, 200, 64

BB = 256
GRID = B // BB


def _body(mask_ref, w_ref, out_ref):
    m = mask_ref[...][:, :, None]
    w = w_ref[...]
    w0 = w[0:1, :].reshape(1, 1, D)
    w1 = w[1:2, :].reshape(1, 1, D)
    w2 = w[2:3, :].reshape(1, 1, D)
    w3 = w[3:4, :].reshape(1, 1, D)
    out_ref[...] = jnp.where(m == 0, w0,
                   jnp.where(m == 1, w1,
                   jnp.where(m == 2, w2, w3)))


def kernel(mask, emb_weight):
    return pl.pallas_call(
        _body,
        grid=(GRID,),
        in_specs=[
            pl.BlockSpec((BB, L), lambda g: (g, 0)),
            pl.BlockSpec((4, D), lambda g: (0, 0)),
        ],
        out_specs=pl.BlockSpec((BB, L, D), lambda g: (g, 0, 0)),
        out_shape=jax.ShapeDtypeStruct((B, L, D), jnp.float32),
    )(mask.astype(jnp.int32), emb_weight)


# speedup vs baseline: 1.0207x; 1.0207x over previous
import jax
import jax.numpy as jnp
from jax.experimental import pallas as pl
from jax.experimental.pallas import tpu as pltpu

B, L, D = 4096, 200, 64

BB = 64
GRID = B // BB
NBUF = 4


def _body(mask_ref, w_ref, out_ref, scratch, sems):
    g = pl.program_id(0)
    slot = jax.lax.rem(g, NBUF)

    @pl.when(g >= NBUF)
    def _():
        pltpu.make_async_copy(
            scratch.at[slot], out_ref.at[pl.ds((g - NBUF) * BB, BB)],
            sems.at[slot]).wait()

    m = mask_ref[...][:, :, None]
    w = w_ref[...]
    w0 = w[0:1, :].reshape(1, 1, D)
    w1 = w[1:2, :].reshape(1, 1, D)
    w2 = w[2:3, :].reshape(1, 1, D)
    w3 = w[3:4, :].reshape(1, 1, D)
    scratch[slot] = jnp.where(m == 0, w0,
                    jnp.where(m == 1, w1,
                    jnp.where(m == 2, w2, w3)))

    pltpu.make_async_copy(
        scratch.at[slot], out_ref.at[pl.ds(g * BB, BB)],
        sems.at[slot]).start()

    @pl.when(g == GRID - 1)
    def _():
        for k in range(NBUF):
            s = jax.lax.rem(g - (NBUF - 1) + k, NBUF)
            pltpu.make_async_copy(
                scratch.at[s],
                out_ref.at[pl.ds((g - (NBUF - 1) + k) * BB, BB)],
                sems.at[s]).wait()


def kernel(mask, emb_weight):
    return pl.pallas_call(
        _body,
        grid=(GRID,),
        in_specs=[
            pl.BlockSpec((BB, L), lambda g: (g, 0)),
            pl.BlockSpec((4, D), lambda g: (0, 0)),
        ],
        out_specs=pl.BlockSpec(memory_space=pl.ANY),
        out_shape=jax.ShapeDtypeStruct((B, L, D), jnp.float32),
        scratch_shapes=[
            pltpu.VMEM((NBUF, BB, L, D), jnp.float32),
            pltpu.SemaphoreType.DMA((NBUF,)),
        ],
    )(mask.astype(jnp.int32), emb_weight)
